# Initial kernel scaffold; baseline (speedup 1.0000x reference)
import jax
import jax.numpy as jnp
from jax.experimental import pallas as pl


def kernel(inputs, embeddings):
    raise NotImplementedError("write your pallas kernel here")


# speedup vs baseline: 1.1080x; 1.1080x over previous
import jax
import jax.numpy as jnp
from jax.experimental import pallas as pl
from jax.experimental.pallas import tpu as pltpu


def _vq_body(x_ref, emb_ref, embt_ref, q_ref, enc_ref, idx_ref, loss_ref):
    i = pl.program_id(0)
    nblk = pl.num_programs(0)
    x = x_ref[...]
    emb = emb_ref[...]
    blk, k = enc_ref.shape

    sim = jnp.dot(x, emb, preferred_element_type=jnp.float32)
    xsq = jnp.sum(x * x, axis=1, keepdims=True)
    esq = jnp.sum(emb * emb, axis=0, keepdims=True)
    neg = 2.0 * sim - xsq - esq

    m = jnp.max(neg, axis=1, keepdims=True)
    iota = jax.lax.broadcasted_iota(jnp.int32, (blk, k), 1)
    idx = jnp.min(jnp.where(neg == m, iota, k), axis=1)

    onehot = (iota == idx[:, None]).astype(jnp.float32)
    enc_ref[...] = onehot
    idx_ref[...] = idx[:, None]

    q = jax.lax.dot_general(onehot, embt_ref[...], (((1,), (0,)), ((), ())),
                            precision=jax.lax.Precision.HIGHEST,
                            preferred_element_type=jnp.float32)
    q_ref[...] = q

    diff = q - x
    part = jnp.sum(diff * diff).reshape(1, 1)

    @pl.when(i == 0)
    def _init():
        loss_ref[...] = jnp.zeros((1, 1), jnp.float32)
    loss_ref[...] += part


def kernel(inputs, embeddings):
    b, t, d = inputs.shape
    k = embeddings.shape[1]
    n = b * t
    blk = 1024
    grid = n // blk

    flat = inputs.reshape(n, d)
    embt = embeddings.T

    q, enc, idx, loss_acc = pl.pallas_call(
        _vq_body,
        grid=(grid,),
        in_specs=[
            pl.BlockSpec((blk, d), lambda i: (i, 0)),
            pl.BlockSpec((d, k), lambda i: (0, 0)),
            pl.BlockSpec((k, d), lambda i: (0, 0)),
        ],
        out_specs=[
            pl.BlockSpec((blk, d), lambda i: (i, 0)),
            pl.BlockSpec((blk, k), lambda i: (i, 0)),
            pl.BlockSpec((blk, 1), lambda i: (i, 0)),
            pl.BlockSpec((1, 1), lambda i: (0, 0)),
        ],
        out_shape=[
            jax.ShapeDtypeStruct((n, d), jnp.float32),
            jax.ShapeDtypeStruct((n, k), jnp.float32),
            jax.ShapeDtypeStruct((n, 1), jnp.int32),
            jax.ShapeDtypeStruct((1, 1), jnp.float32),
        ],
    )(flat, embeddings, embt)

    quantized_st = q.reshape(b, t, d)
    idx_out = idx.reshape(b, t)
    loss = loss_acc[0, 0] * (1.25 / (n * d))
    return quantized_st, enc, idx_out, loss


# speedup vs baseline: 1.6808x; 1.5169x over previous
import jax
import jax.numpy as jnp
from jax.experimental import pallas as pl
from jax.experimental.pallas import tpu as pltpu


def _vq_body(x_ref, emb_ref, embt_ref, q_ref, enc_ref, idx_ref, loss_ref):
    i = pl.program_id(0)
    nblk = pl.num_programs(0)
    x = x_ref[...]
    emb = emb_ref[...]
    blk, k = enc_ref.shape

    sim = jnp.dot(x, emb, preferred_element_type=jnp.float32)
    xsq = jnp.sum(x * x, axis=1, keepdims=True)
    esq = jnp.sum(emb * emb, axis=0, keepdims=True)
    neg = -(xsq - 2.0 * sim + esq)

    idx = jnp.argmax(neg, axis=1).astype(jnp.int32)
    iota = jax.lax.broadcasted_iota(jnp.int32, (blk, k), 1)

    onehot = (iota == idx[:, None]).astype(jnp.float32)
    enc_ref[...] = onehot
    idx_ref[...] = idx[:, None]

    q = jnp.dot(onehot, embt_ref[...], preferred_element_type=jnp.float32)
    q_ref[...] = q

    diff = q - x
    part = jnp.sum(diff * diff).reshape(1, 1)

    @pl.when(i == 0)
    def _init():
        loss_ref[...] = jnp.zeros((1, 1), jnp.float32)
    loss_ref[...] += part


def kernel(inputs, embeddings):
    b, t, d = inputs.shape
    k = embeddings.shape[1]
    n = b * t
    blk = 1024
    grid = n // blk

    flat = inputs.reshape(n, d)
    embt = embeddings.T

    q, enc, idx, loss_acc = pl.pallas_call(
        _vq_body,
        grid=(grid,),
        in_specs=[
            pl.BlockSpec((blk, d), lambda i: (i, 0)),
            pl.BlockSpec((d, k), lambda i: (0, 0)),
            pl.BlockSpec((k, d), lambda i: (0, 0)),
        ],
        out_specs=[
            pl.BlockSpec((blk, d), lambda i: (i, 0)),
            pl.BlockSpec((blk, k), lambda i: (i, 0)),
            pl.BlockSpec((blk, 1), lambda i: (i, 0)),
            pl.BlockSpec((1, 1), lambda i: (0, 0)),
        ],
        out_shape=[
            jax.ShapeDtypeStruct((n, d), jnp.float32),
            jax.ShapeDtypeStruct((n, k), jnp.float32),
            jax.ShapeDtypeStruct((n, 1), jnp.int32),
            jax.ShapeDtypeStruct((1, 1), jnp.float32),
        ],
    )(flat, embeddings, embt)

    quantized_st = q.reshape(b, t, d)
    idx_out = idx.reshape(b, t)
    loss = loss_acc[0, 0] * (1.25 / (n * d))
    return quantized_st, enc, idx_out, loss


# speedup vs baseline: 1.7267x; 1.0273x over previous
import jax
import jax.numpy as jnp
from jax.experimental import pallas as pl
from jax.experimental.pallas import tpu as pltpu


def _vq_body(x_ref, emb_ref, embt_ref, q_ref, enc_ref, idx_ref, loss_ref):
    i = pl.program_id(0)
    nblk = pl.num_programs(0)
    x = x_ref[...]
    emb = emb_ref[...]
    blk, k = enc_ref.shape

    sim = jnp.dot(x, emb, preferred_element_type=jnp.float32)
    xsq = jnp.sum(x * x, axis=1, keepdims=True)
    esq = jnp.sum(emb * emb, axis=0, keepdims=True)
    dist = xsq - 2.0 * sim + esq

    idx = jnp.argmin(dist, axis=1).astype(jnp.int32)
    iota = jax.lax.broadcasted_iota(jnp.int32, (blk, k), 1)

    onehot = (iota == idx[:, None]).astype(jnp.float32)
    enc_ref[...] = onehot
    idx_ref[...] = idx[:, None]

    q = jnp.dot(onehot, embt_ref[...], preferred_element_type=jnp.float32)
    q_ref[...] = q

    diff = q - x
    part = jnp.sum(diff * diff).reshape(1, 1)

    @pl.when(i == 0)
    def _init():
        loss_ref[...] = jnp.zeros((1, 1), jnp.float32)
    loss_ref[...] += part


def kernel(inputs, embeddings):
    b, t, d = inputs.shape
    k = embeddings.shape[1]
    n = b * t
    blk = 1024
    grid = n // blk

    flat = inputs.reshape(n, d)
    embt = embeddings.T

    q, enc, idx, loss_acc = pl.pallas_call(
        _vq_body,
        grid=(grid,),
        in_specs=[
            pl.BlockSpec((blk, d), lambda i: (i, 0)),
            pl.BlockSpec((d, k), lambda i: (0, 0)),
            pl.BlockSpec((k, d), lambda i: (0, 0)),
        ],
        out_specs=[
            pl.BlockSpec((blk, d), lambda i: (i, 0)),
            pl.BlockSpec((blk, k), lambda i: (i, 0)),
            pl.BlockSpec((blk, 1), lambda i: (i, 0)),
            pl.BlockSpec((1, 1), lambda i: (0, 0)),
        ],
        out_shape=[
            jax.ShapeDtypeStruct((n, d), jnp.float32),
            jax.ShapeDtypeStruct((n, k), jnp.float32),
            jax.ShapeDtypeStruct((n, 1), jnp.int32),
            jax.ShapeDtypeStruct((1, 1), jnp.float32),
        ],
    )(flat, embeddings, embt)

    quantized_st = q.reshape(b, t, d)
    idx_out = idx.reshape(b, t)
    loss = loss_acc[0, 0] * (1.25 / (n * d))
    return quantized_st, enc, idx_out, loss


# speedup vs baseline: 1.8577x; 1.0759x over previous
import jax
import jax.numpy as jnp
from jax.experimental import pallas as pl
from jax.experimental.pallas import tpu as pltpu


def _vq_body(x_ref, emb_ref, embt_ref, q_ref, enc_ref, idx_ref, loss_ref):
    i = pl.program_id(0)
    nblk = pl.num_programs(0)
    x = x_ref[...]
    emb = emb_ref[...]
    blk, k = enc_ref.shape

    sim = jnp.dot(x, emb, preferred_element_type=jnp.float32)
    xsq = jnp.sum(x * x, axis=1, keepdims=True)
    esq = jnp.sum(emb * emb, axis=0, keepdims=True)
    dist = xsq - 2.0 * sim + esq

    idx = jnp.argmin(dist, axis=1).astype(jnp.int32)
    iota = jax.lax.broadcasted_iota(jnp.int32, (blk, k), 1)

    onehot = (iota == idx[:, None]).astype(jnp.float32)
    enc_ref[...] = onehot
    idx_ref[...] = idx.reshape(1, 1, blk)

    q = jnp.dot(onehot, embt_ref[...], preferred_element_type=jnp.float32)
    q_ref[...] = q

    diff = q - x
    part = jnp.sum(diff * diff).reshape(1, 1)

    @pl.when(i == 0)
    def _init():
        loss_ref[...] = jnp.zeros((1, 1), jnp.float32)
    loss_ref[...] += part


def kernel(inputs, embeddings):
    b, t, d = inputs.shape
    k = embeddings.shape[1]
    n = b * t
    blk = 1024
    grid = n // blk

    flat = inputs.reshape(n, d)
    embt = embeddings.T

    q, enc, idx, loss_acc = pl.pallas_call(
        _vq_body,
        grid=(grid,),
        in_specs=[
            pl.BlockSpec((blk, d), lambda i: (i, 0)),
            pl.BlockSpec((d, k), lambda i: (0, 0)),
            pl.BlockSpec((k, d), lambda i: (0, 0)),
        ],
        out_specs=[
            pl.BlockSpec((blk, d), lambda i: (i, 0)),
            pl.BlockSpec((blk, k), lambda i: (i, 0)),
            pl.BlockSpec((1, 1, blk), lambda i: (i, 0, 0)),
            pl.BlockSpec((1, 1), lambda i: (0, 0)),
        ],
        out_shape=[
            jax.ShapeDtypeStruct((n, d), jnp.float32),
            jax.ShapeDtypeStruct((n, k), jnp.float32),
            jax.ShapeDtypeStruct((grid, 1, blk), jnp.int32),
            jax.ShapeDtypeStruct((1, 1), jnp.float32),
        ],
    )(flat, embeddings, embt)

    quantized_st = q.reshape(b, t, d)
    idx_out = idx.reshape(b, t)
    loss = loss_acc[0, 0] * (1.25 / (n * d))
    return quantized_st, enc, idx_out, loss


# speedup vs baseline: 1.9998x; 1.0765x over previous
import jax
import jax.numpy as jnp
from jax.experimental import pallas as pl
from jax.experimental.pallas import tpu as pltpu


def _vq_body(x_ref, emb_ref, embt_ref, q_ref, enc_ref, idx_ref, loss_ref):
    i = pl.program_id(0)
    nblk = pl.num_programs(0)
    x = x_ref[...]
    emb = emb_ref[...]
    blk, k = enc_ref.shape

    sim = jnp.dot(x, emb, preferred_element_type=jnp.float32)
    xsq = jnp.sum(x * x, axis=1, keepdims=True)
    esq = jnp.sum(emb * emb, axis=0, keepdims=True)
    dist = xsq - 2.0 * sim + esq

    idx = jnp.argmin(dist, axis=1).astype(jnp.int32)
    iota = jax.lax.broadcasted_iota(jnp.int32, (blk, k), 1)

    onehot = (iota == idx[:, None]).astype(jnp.float32)
    enc_ref[...] = onehot
    idx_ref[...] = idx.reshape(1, 1, blk)

    q = jnp.dot(onehot, embt_ref[...], preferred_element_type=jnp.float32)
    q_ref[...] = q

    diff = q - x
    part = jnp.sum(diff * diff).reshape(1, 1)

    @pl.when(i == 0)
    def _init():
        loss_ref[...] = jnp.zeros((1, 1), jnp.float32)
    loss_ref[...] += part


def kernel(inputs, embeddings):
    b, t, d = inputs.shape
    k = embeddings.shape[1]
    n = b * t
    blk = 2048
    grid = n // blk

    flat = inputs.reshape(n, d)
    embt = embeddings.T

    q, enc, idx, loss_acc = pl.pallas_call(
        _vq_body,
        grid=(grid,),
        in_specs=[
            pl.BlockSpec((blk, d), lambda i: (i, 0)),
            pl.BlockSpec((d, k), lambda i: (0, 0)),
            pl.BlockSpec((k, d), lambda i: (0, 0)),
        ],
        out_specs=[
            pl.BlockSpec((blk, d), lambda i: (i, 0)),
            pl.BlockSpec((blk, k), lambda i: (i, 0)),
            pl.BlockSpec((1, 1, blk), lambda i: (i, 0, 0)),
            pl.BlockSpec((1, 1), lambda i: (0, 0)),
        ],
        out_shape=[
            jax.ShapeDtypeStruct((n, d), jnp.float32),
            jax.ShapeDtypeStruct((n, k), jnp.float32),
            jax.ShapeDtypeStruct((grid, 1, blk), jnp.int32),
            jax.ShapeDtypeStruct((1, 1), jnp.float32),
        ],
    )(flat, embeddings, embt)

    quantized_st = q.reshape(b, t, d)
    idx_out = idx.reshape(b, t)
    loss = loss_acc[0, 0] * (1.25 / (n * d))
    return quantized_st, enc, idx_out, loss
